# baseline (speedup 1.0000x reference)
import functools

import jax
import jax.numpy as jnp
from jax import lax
from jax.experimental import pallas as pl
from jax.experimental.pallas import tpu as pltpu
from jax.experimental.pallas import tpu_sc as plsc

_NROWS = 26
_W = 16384
_Q = _W // 4
_TOTAL_U = _NROWS * 16
_NSLOT = 15
_NACT = 2 * _NSLOT
_KMAX = 14
_NB = 3

_mesh = plsc.VectorSubcoreMesh(core_axis_name="c", subcore_axis_name="s")


@functools.partial(
    pl.kernel,
    out_type=jax.ShapeDtypeStruct((_NROWS * 32, _W), jnp.float32),
    mesh=_mesh,
    scratch_types=[
        pltpu.VMEM((16,), jnp.int32),
        pltpu.VMEM_SHARED((_NSLOT, _NB, 8, _Q), jnp.float32),
        pltpu.SemaphoreType.DMA((_NB,)),
        pltpu.SemaphoreType.DMA((_NB,)),
    ],
)
def _sc_gather(table, idx_hbm, out, idxv, shared, gsem, psem):
    sid = lax.axis_index("s")
    cid = lax.axis_index("c")
    vid = cid * _NSLOT + sid
    active = sid < _NSLOT

    @pl.when(active)
    def _():
        pltpu.sync_copy(idx_hbm.at[vid], idxv)

    srows = idxv[...]

    def unit(j):
        return j * _NACT + vid

    def valid(j):
        if j >= _KMAX:
            return active & False
        return active & (unit(j) < _TOTAL_U)

    def gather(j):
        u = unit(j)
        s = srows[j]
        return pltpu.make_async_copy(
            table.at[pl.ds(s * 8, 8), pl.ds((u % 4) * _Q, _Q)],
            shared.at[sid, j % _NB],
            gsem.at[j % _NB],
        )

    def put(j):
        u = unit(j)
        return pltpu.make_async_copy(
            shared.at[sid, j % _NB],
            out.at[pl.ds((u // 4) * 8, 8), pl.ds((u % 4) * _Q, _Q)],
            psem.at[j % _NB],
        )

    for i in range(_NB - 1):

        @pl.when(valid(i))
        def _(i=i):
            gather(i).start()

    for j in range(_KMAX):

        @pl.when(valid(j))
        def _(j=j):
            gather(j).wait()
            put(j).start()

        if j + _NB - 1 < _KMAX:

            @pl.when(valid(j + _NB - 1))
            def _(j=j):
                if j >= 1:
                    put(j - 1).wait()
                gather(j + _NB - 1).start()

    for j in range(_KMAX - _NB - 1, _KMAX):
        if j < 0:
            continue

        @pl.when(valid(j) & ~valid(j + _NB))
        def _(j=j):
            put(j).wait()


def kernel(mamdani_output, mapping):
    src = jnp.transpose(mamdani_output, (0, 2, 1)).reshape(3200, _W)
    m = mapping.reshape(_NROWS)
    j = jnp.arange(16, dtype=jnp.int32)[None, :]
    v = jnp.arange(32, dtype=jnp.int32)[:, None]
    u = jnp.minimum(j * _NACT + v, _TOTAL_U - 1)
    tr = u // 4
    idx = m[tr // 4] * 4 + (tr % 4)
    out = _sc_gather(src, idx)
    out = jnp.transpose(out.reshape(_NROWS, 32, _W), (0, 2, 1))
    return jnp.expand_dims(out, 1)
